# baseline (speedup 1.0000x reference)
import functools

import jax
import jax.numpy as jnp
from jax import lax
from jax.experimental import pallas as pl
from jax.experimental.pallas import tpu as pltpu
from jax.experimental.pallas import tpu_sc as plsc

B = 4
SEQ = 4096
D = 2048
NUM_BUCKETS = 64
MAX_SKIPPED_FLOPS = float(12 * (12 * D * D * SEQ))

R = B * SEQ
NC = 2
NS = 16
NW = NC * NS
C = 8

RT = 10240
RS = R - RT
RPW = RS // NW
NCH = RPW // C

BR = 512

_mesh = plsc.VectorSubcoreMesh(core_axis_name="c", subcore_axis_name="s")


def _bucketize(v):
    bkt = ((v / MAX_SKIPPED_FLOPS) * float(NUM_BUCKETS)).astype(jnp.int32)
    return jnp.minimum(jnp.maximum(bkt, 0), NUM_BUCKETS - 1)


@functools.partial(
    pl.kernel,
    mesh=_mesh,
    out_type=jax.ShapeDtypeStruct((RS, D), jnp.float32),
    scratch_types=[
        pltpu.VMEM((RPW,), jnp.float32),
        pltpu.VMEM((RPW,), jnp.int32),
        pltpu.VMEM((C, D), jnp.float32),
        pltpu.VMEM((C, D), jnp.float32),
        pltpu.VMEM((C, D), jnp.float32),
        pltpu.VMEM((C, D), jnp.float32),
        pltpu.VMEM((C, D), jnp.float32),
        pltpu.VMEM((C, D), jnp.float32),
        pltpu.SemaphoreType.DMA,
        pltpu.SemaphoreType.DMA,
        pltpu.SemaphoreType.DMA,
        pltpu.SemaphoreType.DMA,
        pltpu.SemaphoreType.DMA,
        pltpu.SemaphoreType.DMA,
        pltpu.SemaphoreType.DMA,
        pltpu.SemaphoreType.DMA,
        pltpu.SemaphoreType.DMA,
        pltpu.SemaphoreType.DMA,
    ],
)
def _sc_step_encoding(x_hbm, csf_hbm, table_hbm, out_hbm,
                      csf_v, idx_v, xb0, xb1, eb0, eb1, eb2, eb3,
                      xs0, xs1, gs0, gs1, gs2, gs3, os0, os1, os2, os3):
    xbufs = (xb0, xb1)
    ebufs = (eb0, eb1, eb2, eb3)
    xsems = (xs0, xs1)
    gsems = (gs0, gs1, gs2, gs3)
    osems = (os0, os1, os2, os3)

    wid = lax.axis_index("s") * NC + lax.axis_index("c")
    base = RT + wid * RPW

    pltpu.sync_copy(csf_hbm.at[pl.ds(base, RPW)], csf_v)

    def idx_body(j):
        idx_v[pl.ds(j * 16, 16)] = _bucketize(csf_v[pl.ds(j * 16, 16)])
    pl.loop(0, RPW // 16)(idx_body)

    def xrows(ci):
        return pl.ds(base + ci * C, C)

    def orows(ci):
        return pl.ds(base - RT + ci * C, C)

    def start_x(ci, s):
        pltpu.async_copy(x_hbm.at[xrows(ci)], xbufs[s], xsems[s])

    def wait_x(ci, s):
        pltpu.make_async_copy(x_hbm.at[xrows(ci)], xbufs[s], xsems[s]).wait()

    def start_g(ci, s):
        pltpu.async_copy(
            table_hbm.at[idx_v.at[pl.ds(ci * C, C)]], ebufs[s], gsems[s])

    def wait_g(ci, s):
        pltpu.make_async_copy(
            table_hbm.at[idx_v.at[pl.ds(ci * C, C)]], ebufs[s],
            gsems[s]).wait()

    def start_o(ci, s):
        pltpu.async_copy(ebufs[s], out_hbm.at[orows(ci)], osems[s])

    def wait_o(ci, s):
        pltpu.make_async_copy(ebufs[s], out_hbm.at[orows(ci)],
                              osems[s]).wait()

    def add_chunk(xs, es):
        xb = xbufs[xs]
        eb = ebufs[es]

        def row_body(r):
            @plsc.parallel_loop(0, D // 16, unroll=8)
            def col_body(k):
                off = k * 16
                xv = xb[r, pl.ds(off, 16)]
                plsc.addupdate(eb.at[r, pl.ds(off, 16)], xv)
        pl.loop(0, C)(row_body)

    def proc(ci, u, first, last):
        xs, es = u % 2, u % 4
        wait_x(ci, xs)
        wait_g(ci, es)
        add_chunk(xs, es)
        start_o(ci, es)
        if not last:
            start_x(ci + 2, xs)
        if not first:
            wait_o(ci - 2, (es + 2) % 4)
        if not last:
            start_g(ci + 2, (es + 2) % 4)

    start_x(0, 0)
    start_x(1, 1)
    start_g(0, 0)
    start_g(1, 1)
    for u in range(4):
        proc(u, u, first=(u < 2), last=False)

    def quad_body(q):
        for u in range(4):
            proc(q * 4 + u, u, first=False, last=False)
    pl.loop(1, NCH // 4 - 1)(quad_body)

    for u in range(4):
        proc(NCH - 4 + u, u, first=False, last=(u >= 2))
    for u in (2, 3):
        wait_o(NCH - 4 + u, u)


def _tc_block(x_ref, csf_ref, table_ref, out_ref):
    idx = _bucketize(csf_ref[0, 0, :])
    onehot = (idx[:, None]
              == lax.broadcasted_iota(jnp.int32, (1, NUM_BUCKETS), 1)
              ).astype(jnp.float32)
    emb = jnp.dot(onehot, table_ref[...],
                  preferred_element_type=jnp.float32)
    out_ref[...] = x_ref[...] + emb


def _tc_call(x2d, csf3d, table):
    return pl.pallas_call(
        _tc_block,
        grid=(RT // BR,),
        in_specs=[
            pl.BlockSpec((BR, D), lambda i: (i, 0)),
            pl.BlockSpec((1, 1, BR), lambda i: (i, 0, 0)),
            pl.BlockSpec((NUM_BUCKETS, D), lambda i: (0, 0)),
        ],
        out_specs=pl.BlockSpec((BR, D), lambda i: (i, 0)),
        out_shape=jax.ShapeDtypeStruct((RT, D), jnp.float32),
    )(x2d, csf3d, table)


def kernel(x, cumulative_skipped_flops, step_embeddings_weight):
    x2 = x.reshape(R, D)
    c = cumulative_skipped_flops.reshape(R)
    tc_out = _tc_call(x2, c.reshape(R // BR, 1, BR), step_embeddings_weight)
    sc_out = _sc_step_encoding(x2, c, step_embeddings_weight)
    return jnp.concatenate([tc_out, sc_out], axis=0).reshape(B, SEQ, D)


# speedup vs baseline: 1.5774x; 1.5774x over previous
import functools

import jax
import jax.numpy as jnp
from jax import lax
from jax.experimental import pallas as pl
from jax.experimental.pallas import tpu as pltpu
from jax.experimental.pallas import tpu_sc as plsc

B = 4
SEQ = 4096
D = 2048
NUM_BUCKETS = 64
MAX_SKIPPED_FLOPS = float(12 * (12 * D * D * SEQ))

R = B * SEQ
NC = 2
NS = 16
NW = NC * NS
RPW = R // NW
C = 8
NCH = RPW // C

_mesh = plsc.VectorSubcoreMesh(core_axis_name="c", subcore_axis_name="s")


def _bucketize(v):
    bkt = ((v / MAX_SKIPPED_FLOPS) * float(NUM_BUCKETS)).astype(jnp.int32)
    return jnp.minimum(jnp.maximum(bkt, 0), NUM_BUCKETS - 1)


@functools.partial(
    pl.kernel,
    mesh=_mesh,
    out_type=jax.ShapeDtypeStruct((R, D), jnp.float32),
    scratch_types=[
        pltpu.VMEM_SHARED((NUM_BUCKETS, D), jnp.float32),
        pltpu.VMEM((RPW,), jnp.float32),
        pltpu.VMEM((RPW + 16,), jnp.int32),
        pltpu.VMEM((C, D), jnp.float32),
        pltpu.VMEM((C, D), jnp.float32),
        pltpu.VMEM((C, D), jnp.float32),
        pltpu.VMEM((C, D), jnp.float32),
        pltpu.VMEM((C, D), jnp.float32),
        pltpu.VMEM((C, D), jnp.float32),
        pltpu.SemaphoreType.DMA,
        pltpu.SemaphoreType.DMA,
        pltpu.SemaphoreType.DMA,
        pltpu.SemaphoreType.DMA,
        pltpu.SemaphoreType.DMA,
        pltpu.SemaphoreType.DMA,
        pltpu.SemaphoreType.DMA,
        pltpu.SemaphoreType.DMA,
        pltpu.SemaphoreType.DMA,
        pltpu.SemaphoreType.DMA,
    ],
)
def _sc_step_encoding(x_hbm, csf_hbm, table_hbm, out_hbm,
                      table_spm, csf_v, idx_v,
                      xb0, xb1, eb0, eb1, eb2, eb3,
                      xs0, xs1, gs0, gs1, gs2, gs3, os0, os1, os2, os3):
    xbufs = (xb0, xb1)
    ebufs = (eb0, eb1, eb2, eb3)
    xsems = (xs0, xs1)
    gsems = (gs0, gs1, gs2, gs3)
    osems = (os0, os1, os2, os3)

    wid = lax.axis_index("s") * NC + lax.axis_index("c")
    base = wid * RPW

    @pl.when(lax.axis_index("s") == 0)
    def _():
        pltpu.sync_copy(table_hbm, table_spm)

    pltpu.sync_copy(csf_hbm.at[pl.ds(base, RPW)], csf_v)

    def idx_body(j):
        idx_v[pl.ds(j * 16, 16)] = _bucketize(csf_v[pl.ds(j * 16, 16)])
    pl.loop(0, RPW // 16)(idx_body)

    plsc.subcore_barrier()

    def rows(ci):
        return pl.ds(base + ci * C, C)

    def start_x(ci, s):
        pltpu.async_copy(x_hbm.at[rows(ci)], xbufs[s], xsems[s])

    def wait_x(ci, s):
        pltpu.make_async_copy(x_hbm.at[rows(ci)], xbufs[s], xsems[s]).wait()

    def start_g(ci, s):
        v16 = idx_v[pl.ds(ci * C, 16)]
        for r in range(C):
            pltpu.async_copy(table_spm.at[pl.ds(v16[r], 1)],
                             ebufs[s].at[pl.ds(r, 1)], gsems[s])

    def wait_g(ci, s):
        pltpu.make_async_copy(table_spm.at[pl.ds(0, C)], ebufs[s],
                              gsems[s]).wait()

    def start_o(ci, s):
        pltpu.async_copy(ebufs[s], out_hbm.at[rows(ci)], osems[s])

    def wait_o(ci, s):
        pltpu.make_async_copy(ebufs[s], out_hbm.at[rows(ci)],
                              osems[s]).wait()

    def add_chunk(xs, es):
        xb = xbufs[xs]
        eb = ebufs[es]

        def row_body(r):
            @plsc.parallel_loop(0, D // 16, unroll=8)
            def col_body(k):
                off = k * 16
                xv = xb[r, pl.ds(off, 16)]
                plsc.addupdate(eb.at[r, pl.ds(off, 16)], xv)
        pl.loop(0, C)(row_body)

    def proc(ci, u, first, last):
        xs, es = u % 2, u % 4
        wait_x(ci, xs)
        wait_g(ci, es)
        add_chunk(xs, es)
        start_o(ci, es)
        if not last:
            start_x(ci + 2, xs)
        if not first:
            wait_o(ci - 2, (es + 2) % 4)
        if not last:
            start_g(ci + 2, (es + 2) % 4)

    start_x(0, 0)
    start_x(1, 1)
    start_g(0, 0)
    start_g(1, 1)
    for u in range(4):
        proc(u, u, first=(u < 2), last=False)

    def quad_body(q):
        for u in range(4):
            proc(q * 4 + u, u, first=False, last=False)
    pl.loop(1, NCH // 4 - 1)(quad_body)

    for u in range(4):
        proc(NCH - 4 + u, u, first=False, last=(u >= 2))
    for u in (2, 3):
        wait_o(NCH - 4 + u, u)


def kernel(x, cumulative_skipped_flops, step_embeddings_weight):
    out = _sc_step_encoding(
        x.reshape(R, D),
        cumulative_skipped_flops.reshape(R),
        step_embeddings_weight,
    )
    return out.reshape(B, SEQ, D)


# speedup vs baseline: 1.6018x; 1.0155x over previous
import functools

import jax
import jax.numpy as jnp
from jax import lax
from jax.experimental import pallas as pl
from jax.experimental.pallas import tpu as pltpu
from jax.experimental.pallas import tpu_sc as plsc

B = 4
SEQ = 4096
D = 2048
NUM_BUCKETS = 64
MAX_SKIPPED_FLOPS = float(12 * (12 * D * D * SEQ))

R = B * SEQ
NC = 2
NS = 16
NW = NC * NS
RPW = R // NW
C = 8
NCH = RPW // C

_mesh = plsc.VectorSubcoreMesh(core_axis_name="c", subcore_axis_name="s")


def _bucketize(v):
    bkt = ((v / MAX_SKIPPED_FLOPS) * float(NUM_BUCKETS)).astype(jnp.int32)
    return jnp.minimum(jnp.maximum(bkt, 0), NUM_BUCKETS - 1)


@functools.partial(
    pl.kernel,
    mesh=_mesh,
    out_type=jax.ShapeDtypeStruct((R, D), jnp.float32),
    scratch_types=[
        pltpu.VMEM_SHARED((NUM_BUCKETS, D), jnp.float32),
        pltpu.VMEM((RPW,), jnp.float32),
        pltpu.VMEM((RPW + 16,), jnp.int32),
        pltpu.VMEM((C, D), jnp.float32),
        pltpu.VMEM((C, D), jnp.float32),
        pltpu.VMEM((C, D), jnp.float32),
        pltpu.VMEM((C, D), jnp.float32),
        pltpu.VMEM((C, D), jnp.float32),
        pltpu.VMEM((C, D), jnp.float32),
        pltpu.SemaphoreType.DMA,
        pltpu.SemaphoreType.DMA,
        pltpu.SemaphoreType.DMA,
        pltpu.SemaphoreType.DMA,
        pltpu.SemaphoreType.DMA,
        pltpu.SemaphoreType.DMA,
        pltpu.SemaphoreType.DMA,
        pltpu.SemaphoreType.DMA,
        pltpu.SemaphoreType.DMA,
        pltpu.SemaphoreType.DMA,
    ],
)
def _sc_step_encoding(x_hbm, csf_hbm, table_hbm, out_hbm,
                      table_spm, csf_v, idx_v,
                      xb0, xb1, eb0, eb1, eb2, eb3,
                      xs0, xs1, gs0, gs1, gs2, gs3, os0, os1, os2, os3):
    xbufs = (xb0, xb1)
    ebufs = (eb0, eb1, eb2, eb3)
    xsems = (xs0, xs1)
    gsems = (gs0, gs1, gs2, gs3)
    osems = (os0, os1, os2, os3)

    wid = lax.axis_index("s") * NC + lax.axis_index("c")
    base = wid * RPW

    @pl.when(lax.axis_index("s") == 0)
    def _():
        pltpu.sync_copy(table_hbm, table_spm)

    pltpu.sync_copy(csf_hbm.at[pl.ds(base, RPW)], csf_v)

    def idx_body(j):
        idx_v[pl.ds(j * 16, 16)] = _bucketize(csf_v[pl.ds(j * 16, 16)])
    pl.loop(0, RPW // 16)(idx_body)

    plsc.subcore_barrier()

    def rows(ci):
        return pl.ds(base + ci * C, C)

    def start_x(ci, s):
        pltpu.async_copy(x_hbm.at[rows(ci)], xbufs[s], xsems[s])

    def wait_x(ci, s):
        pltpu.make_async_copy(x_hbm.at[rows(ci)], xbufs[s], xsems[s]).wait()

    def start_g(ci, s):
        v16 = idx_v[pl.ds(ci * C, 16)]
        for r in range(C):
            pltpu.async_copy(table_spm.at[pl.ds(v16[r], 1)],
                             ebufs[s].at[pl.ds(r, 1)], gsems[s])

    def wait_g(ci, s):
        pltpu.make_async_copy(table_spm.at[pl.ds(0, C)], ebufs[s],
                              gsems[s]).wait()

    def start_o(ci, s):
        pltpu.async_copy(ebufs[s], out_hbm.at[rows(ci)], osems[s])

    def wait_o(ci, s):
        pltpu.make_async_copy(ebufs[s], out_hbm.at[rows(ci)],
                              osems[s]).wait()

    def add_chunk(xs, es):
        xb = xbufs[xs]
        eb = ebufs[es]

        @plsc.parallel_loop(0, C * (D // 16), unroll=8)
        def col_body(k):
            r = lax.shift_right_logical(k, 7)
            off = pl.multiple_of(
                lax.shift_left(jnp.bitwise_and(k, D // 16 - 1), 4), 16)
            xv = xb[r, pl.ds(off, 16)]
            plsc.addupdate(eb.at[r, pl.ds(off, 16)], xv)

    def proc(ci, u, first, last):
        xs, es = u % 2, u % 4
        wait_x(ci, xs)
        wait_g(ci, es)
        add_chunk(xs, es)
        start_o(ci, es)
        if not last:
            start_x(ci + 2, xs)
        if not first:
            wait_o(ci - 2, (es + 2) % 4)
        if not last:
            start_g(ci + 2, (es + 2) % 4)

    start_x(0, 0)
    start_x(1, 1)
    start_g(0, 0)
    start_g(1, 1)
    for u in range(4):
        proc(u, u, first=(u < 2), last=False)

    def quad_body(q):
        for u in range(4):
            proc(q * 4 + u, u, first=False, last=False)
    pl.loop(1, NCH // 4 - 1)(quad_body)

    for u in range(4):
        proc(NCH - 4 + u, u, first=False, last=(u >= 2))
    for u in (2, 3):
        wait_o(NCH - 4 + u, u)


def kernel(x, cumulative_skipped_flops, step_embeddings_weight):
    out = _sc_step_encoding(
        x.reshape(R, D),
        cumulative_skipped_flops.reshape(R),
        step_embeddings_weight,
    )
    return out.reshape(B, SEQ, D)
